# Initial kernel scaffold; baseline (speedup 1.0000x reference)
import functools

import jax
import jax.numpy as jnp
from jax import lax
from jax.experimental import pallas as pl
from jax.experimental.pallas import tpu as pltpu
from jax.experimental.pallas import tpu_sc as plsc

N_NODES = 10000
N_EDGES = 320000
IN_DIM = 128
HID = 64
NUM_GRAPHS = 128
N_HEADS = 5

NC = 2
NS = 16
NW = NC * NS
EC = 128
EK = 80
E_PAD = NW * EK * EC
NR = 10240
GARBAGE_ROW = 10200

PK = 5
PC = 64
POOL_W = HID + 16
POOL_R = 256

_vmesh = plsc.VectorSubcoreMesh(core_axis_name="c", subcore_axis_name="s")


def _worker_id():
    return lax.axis_index("c") * NS + lax.axis_index("s")


def _zero_shared_slab(zbuf, acc_sh, rows_per_copy, copies_per_sub):
    sid = lax.axis_index("s")
    width = zbuf.shape[-1]

    @pl.loop(0, zbuf.shape[0])
    def _(i):
        @pl.loop(0, width // 16)
        def _(c):
            zbuf[i, pl.ds(c * 16, 16)] = jnp.zeros((16,), jnp.float32)

    base = sid * rows_per_copy * copies_per_sub

    @pl.loop(0, copies_per_sub)
    def _(t):
        pltpu.sync_copy(zbuf, acc_sh.at[pl.ds(base + t * rows_per_copy,
                                              rows_per_copy)])


@functools.partial(
    pl.kernel,
    out_type=jax.ShapeDtypeStruct((NC, NR, 16), jnp.float32),
    mesh=_vmesh,
    scratch_types=[
        pltpu.VMEM((EK, EC), jnp.int32),
        pltpu.VMEM((EC, 16), jnp.float32),
        pltpu.VMEM((128, 16), jnp.float32),
        pltpu.VMEM_SHARED((NR, 16), jnp.float32),
    ],
)
def _deg_sc(dst_hbm, out_hbm, idx_v, ones_v, zbuf, acc_sh):
    cid = lax.axis_index("c")
    sid = lax.axis_index("s")
    wid = _worker_id()

    @pl.loop(0, EC)
    def _(i):
        ones_v[i, :] = jnp.ones((16,), jnp.float32)

    _zero_shared_slab(zbuf, acc_sh, 128, NR // (128 * NS))
    plsc.subcore_barrier()

    pltpu.sync_copy(dst_hbm.at[wid], idx_v)

    @pl.loop(0, EK)
    def _(j):
        pltpu.sync_copy(ones_v, acc_sh.at[idx_v.at[j]], add=True)

    plsc.subcore_barrier()
    rows = NR // NS
    pltpu.sync_copy(acc_sh.at[pl.ds(sid * rows, rows)],
                    out_hbm.at[cid, pl.ds(sid * rows, rows)])


@functools.partial(
    pl.kernel,
    out_type=jax.ShapeDtypeStruct((NC, NR, HID), jnp.float32),
    mesh=_vmesh,
    scratch_types=[
        pltpu.VMEM((EK, EC), jnp.int32),
        pltpu.VMEM((EK, EC), jnp.int32),
        pltpu.VMEM((EC, HID), jnp.float32),
        pltpu.VMEM((EC, HID), jnp.float32),
        pltpu.VMEM((128, HID), jnp.float32),
        pltpu.VMEM_SHARED((NR, HID), jnp.float32),
        pltpu.SemaphoreType.DMA,
    ],
)
def _edge_sc(hp_hbm, src_hbm, dst_hbm, out_hbm,
             idxs_v, idxd_v, rows0, rows1, zbuf, acc_sh, sem):
    cid = lax.axis_index("c")
    sid = lax.axis_index("s")
    wid = _worker_id()

    _zero_shared_slab(zbuf, acc_sh, 128, NR // (128 * NS))
    plsc.subcore_barrier()

    pltpu.sync_copy(src_hbm.at[wid], idxs_v)
    pltpu.sync_copy(dst_hbm.at[wid], idxd_v)

    pltpu.async_copy(hp_hbm.at[idxs_v.at[0]], rows0, sem)

    @pl.loop(0, EK, step=2)
    def _(j):
        pltpu.make_async_copy(hp_hbm.at[idxs_v.at[j]], rows0, sem).wait()
        pltpu.async_copy(hp_hbm.at[idxs_v.at[j + 1]], rows1, sem)
        pltpu.sync_copy(rows0, acc_sh.at[idxd_v.at[j]], add=True)

        pltpu.make_async_copy(hp_hbm.at[idxs_v.at[j + 1]], rows1, sem).wait()

        @pl.when(j + 2 < EK)
        def _():
            pltpu.async_copy(hp_hbm.at[idxs_v.at[j + 2]], rows0, sem)

        pltpu.sync_copy(rows1, acc_sh.at[idxd_v.at[j + 1]], add=True)

    plsc.subcore_barrier()
    rows = NR // NS
    pltpu.sync_copy(acc_sh.at[pl.ds(sid * rows, rows)],
                    out_hbm.at[cid, pl.ds(sid * rows, rows)])


@functools.partial(
    pl.kernel,
    out_type=jax.ShapeDtypeStruct((NC, POOL_R, POOL_W), jnp.float32),
    mesh=_vmesh,
    scratch_types=[
        pltpu.VMEM((PK, PC), jnp.int32),
        pltpu.VMEM((PK, PC, POOL_W), jnp.float32),
        pltpu.VMEM((16, POOL_W), jnp.float32),
        pltpu.VMEM_SHARED((POOL_R, POOL_W), jnp.float32),
    ],
)
def _pool_sc(h_hbm, batch_hbm, out_hbm, idx_v, rows_v, zbuf, acc_sh):
    cid = lax.axis_index("c")
    sid = lax.axis_index("s")
    wid = _worker_id()

    _zero_shared_slab(zbuf, acc_sh, 16, POOL_R // (16 * NS))
    plsc.subcore_barrier()

    pltpu.sync_copy(batch_hbm.at[wid], idx_v)
    pltpu.sync_copy(h_hbm.at[wid], rows_v)

    @pl.loop(0, PK)
    def _(j):
        pltpu.sync_copy(rows_v.at[j], acc_sh.at[idx_v.at[j]], add=True)

    plsc.subcore_barrier()
    rows = POOL_R // NS
    pltpu.sync_copy(acc_sh.at[pl.ds(sid * rows, rows)],
                    out_hbm.at[cid, pl.ds(sid * rows, rows)])


def _mm0_body(x_ref, w_ref, o_ref):
    o_ref[...] = jnp.dot(x_ref[...], w_ref[...],
                         preferred_element_type=jnp.float32)


_mm0 = pl.pallas_call(
    _mm0_body,
    out_shape=jax.ShapeDtypeStruct((N_NODES, HID), jnp.float32),
)


def _k1_body(dp_ref, xw_ref, hp_ref, dinv_ref):
    deg = dp_ref[0, 0:N_NODES, 0:1] + dp_ref[1, 0:N_NODES, 0:1] + 1.0
    dinv = lax.rsqrt(deg)
    dinv_ref[...] = dinv
    hp_ref[...] = xw_ref[...] * dinv


_k1 = pl.pallas_call(
    _k1_body,
    out_shape=(
        jax.ShapeDtypeStruct((N_NODES, HID), jnp.float32),
        jax.ShapeDtypeStruct((N_NODES, 1), jnp.float32),
    ),
)


def _gcn_bn_relu(accp_ref, hp_ref, dinv_ref, b_ref, g_ref, be_ref):
    acc = accp_ref[0, 0:N_NODES, :] + accp_ref[1, 0:N_NODES, :] + hp_ref[...]
    gcn = acc * dinv_ref[...] + b_ref[...]
    m = jnp.mean(gcn, axis=0, keepdims=True)
    c = gcn - m
    v = jnp.mean(c * c, axis=0, keepdims=True)
    return jnp.maximum(g_ref[...] * c * lax.rsqrt(v + 1e-5) + be_ref[...], 0.0)


def _post_body(accp_ref, hp_ref, dinv_ref, b_ref, g_ref, be_ref, wn_ref,
               o_ref):
    h = _gcn_bn_relu(accp_ref, hp_ref, dinv_ref, b_ref, g_ref, be_ref)
    o_ref[...] = jnp.dot(h, wn_ref[...],
                         preferred_element_type=jnp.float32) * dinv_ref[...]


_post = pl.pallas_call(
    _post_body,
    out_shape=jax.ShapeDtypeStruct((N_NODES, HID), jnp.float32),
)


def _post3_body(accp_ref, hp_ref, dinv_ref, b_ref, g_ref, be_ref, o_ref):
    h = _gcn_bn_relu(accp_ref, hp_ref, dinv_ref, b_ref, g_ref, be_ref)
    aug = jnp.concatenate([h, jnp.ones((N_NODES, 16), jnp.float32)], axis=1)
    o_ref[pl.ds(0, N_NODES), :] = aug
    o_ref[pl.ds(N_NODES, NR - N_NODES), :] = jnp.zeros(
        (NR - N_NODES, POOL_W), jnp.float32)


_post3 = pl.pallas_call(
    _post3_body,
    out_shape=jax.ShapeDtypeStruct((NR, POOL_W), jnp.float32),
)


def _head_body(pp_ref, w1_ref, b1_ref, w2t_ref, b2_ref, o_ref):
    s = pp_ref[0, 0:NUM_GRAPHS, :] + pp_ref[1, 0:NUM_GRAPHS, :]
    sums = s[:, 0:HID]
    counts = s[:, HID:HID + 1]
    pooled = sums / jnp.maximum(counts, 1.0)
    z = jnp.maximum(
        jnp.dot(pooled, w1_ref[...], preferred_element_type=jnp.float32)
        + b1_ref[...], 0.0)
    o_ref[...] = lax.dot_general(
        w2t_ref[...], z, (((1,), (1,)), ((), ())),
        preferred_element_type=jnp.float32) + b2_ref[...]


_head = pl.pallas_call(
    _head_body,
    out_shape=jax.ShapeDtypeStruct((N_HEADS, NUM_GRAPHS), jnp.float32),
)


def kernel(x, edge_index, edge_attr, batch,
           W1, b1, g1, be1, W2, b2, g2, be2, W3, b3, g3, be3,
           Wh1, bh1, Wh2, bh2):
    del edge_attr

    src = edge_index[0]
    dst = edge_index[1]
    srcp = jnp.pad(src, (0, E_PAD - N_EDGES)).reshape(NW, EK, EC)
    dstp = jnp.pad(dst, (0, E_PAD - N_EDGES),
                   constant_values=GARBAGE_ROW).reshape(NW, EK, EC)
    batchp = jnp.pad(batch, (0, NR - N_NODES),
                     constant_values=NUM_GRAPHS).reshape(NW, PK, PC)

    b1r, g1r, be1r = b1.reshape(1, HID), g1.reshape(1, HID), be1.reshape(1, HID)
    b2r, g2r, be2r = b2.reshape(1, HID), g2.reshape(1, HID), be2.reshape(1, HID)
    b3r, g3r, be3r = b3.reshape(1, HID), g3.reshape(1, HID), be3.reshape(1, HID)
    w1h = Wh1.transpose(1, 0, 2).reshape(HID, N_HEADS * 32)
    b1h = bh1.reshape(1, N_HEADS * 32)
    w2 = Wh2[:, :, 0]
    w2t = (jnp.eye(N_HEADS, dtype=jnp.float32)[:, :, None]
           * w2[:, None, :]).reshape(N_HEADS, N_HEADS * 32)
    b2h = bh2[:, 0:1]

    degp = _deg_sc(dstp)
    xw = _mm0(x, W1)
    h1p, dinv = _k1(degp, xw)

    acc1 = _edge_sc(h1p, srcp, dstp)
    h2p = _post(acc1, h1p, dinv, b1r, g1r, be1r, W2)
    acc2 = _edge_sc(h2p, srcp, dstp)
    h3p = _post(acc2, h2p, dinv, b2r, g2r, be2r, W3)
    acc3 = _edge_sc(h3p, srcp, dstp)
    h3aug = _post3(acc3, h3p, dinv, b3r, g3r, be3r)

    pp = _pool_sc(h3aug.reshape(NW, PK, PC, POOL_W), batchp)
    return _head(pp, w1h, b1h, w2t, b2h)


# speedup vs baseline: 12.0872x; 12.0872x over previous
import functools

import jax
import jax.numpy as jnp
from jax import lax
from jax.experimental import pallas as pl
from jax.experimental.pallas import tpu as pltpu
from jax.experimental.pallas import tpu_sc as plsc

N_NODES = 10000
N_EDGES = 320000
IN_DIM = 128
HID = 64
NUM_GRAPHS = 128
N_HEADS = 5

NC = 2
NS = 16
NW = NC * NS
EC = 128
EK = 80
E_PAD = NW * EK * EC
NR = 10240
GARBAGE_ROW = 10200

PK = 5
PC = 64
POOL_W = HID + 16
POOL_R = 256

_vmesh = plsc.VectorSubcoreMesh(core_axis_name="c", subcore_axis_name="s")
_sc_params = pltpu.CompilerParams(use_tc_tiling_on_sc=False)


def _worker_id():
    return lax.axis_index("c") * NS + lax.axis_index("s")


def _zero_shared_slab(zbuf, acc_sh, rows_per_copy, copies_per_sub):
    sid = lax.axis_index("s")
    width = zbuf.shape[-1]

    @pl.loop(0, zbuf.shape[0])
    def _(i):
        @pl.loop(0, width // 16)
        def _(c):
            zbuf[i, pl.ds(c * 16, 16)] = jnp.zeros((16,), jnp.float32)

    base = sid * rows_per_copy * copies_per_sub

    @pl.loop(0, copies_per_sub)
    def _(t):
        pltpu.sync_copy(zbuf, acc_sh.at[pl.ds(base + t * rows_per_copy,
                                              rows_per_copy)])


@functools.partial(
    pl.kernel,
    out_type=jax.ShapeDtypeStruct((NC, NR, 16), jnp.float32),
    mesh=_vmesh,
    scratch_types=[
        pltpu.VMEM((EK, EC), jnp.int32),
        pltpu.VMEM((EC, 16), jnp.float32),
        pltpu.VMEM((128, 16), jnp.float32),
        pltpu.VMEM_SHARED((NR, 16), jnp.float32),
    ],
    compiler_params=_sc_params,
)
def _deg_sc(dst_hbm, out_hbm, idx_v, ones_v, zbuf, acc_sh):
    cid = lax.axis_index("c")
    sid = lax.axis_index("s")
    wid = _worker_id()

    @pl.loop(0, EC)
    def _(i):
        ones_v[i, :] = jnp.ones((16,), jnp.float32)

    _zero_shared_slab(zbuf, acc_sh, 128, NR // (128 * NS))
    plsc.subcore_barrier()

    pltpu.sync_copy(dst_hbm.at[wid], idx_v)

    @pl.loop(0, EK)
    def _(j):
        pltpu.sync_copy(ones_v, acc_sh.at[idx_v.at[j]], add=True)

    plsc.subcore_barrier()
    rows = NR // NS
    pltpu.sync_copy(acc_sh.at[pl.ds(sid * rows, rows)],
                    out_hbm.at[cid, pl.ds(sid * rows, rows)])


@functools.partial(
    pl.kernel,
    out_type=jax.ShapeDtypeStruct((NC, NR, HID), jnp.float32),
    mesh=_vmesh,
    scratch_types=[
        pltpu.VMEM((EK, EC), jnp.int32),
        pltpu.VMEM((EK, EC), jnp.int32),
        pltpu.VMEM((EC, HID), jnp.float32),
        pltpu.VMEM((EC, HID), jnp.float32),
        pltpu.VMEM((128, HID), jnp.float32),
        pltpu.VMEM_SHARED((NR, HID), jnp.float32),
        pltpu.SemaphoreType.DMA,
    ],
    compiler_params=_sc_params,
)
def _edge_sc(hp_hbm, src_hbm, dst_hbm, out_hbm,
             idxs_v, idxd_v, rows0, rows1, zbuf, acc_sh, sem):
    cid = lax.axis_index("c")
    sid = lax.axis_index("s")
    wid = _worker_id()

    _zero_shared_slab(zbuf, acc_sh, 128, NR // (128 * NS))
    plsc.subcore_barrier()

    pltpu.sync_copy(src_hbm.at[wid], idxs_v)
    pltpu.sync_copy(dst_hbm.at[wid], idxd_v)

    pltpu.async_copy(hp_hbm.at[idxs_v.at[0]], rows0, sem)

    @pl.loop(0, EK, step=2)
    def _(j):
        pltpu.make_async_copy(hp_hbm.at[idxs_v.at[j]], rows0, sem).wait()
        pltpu.async_copy(hp_hbm.at[idxs_v.at[j + 1]], rows1, sem)
        pltpu.sync_copy(rows0, acc_sh.at[idxd_v.at[j]], add=True)

        pltpu.make_async_copy(hp_hbm.at[idxs_v.at[j + 1]], rows1, sem).wait()

        @pl.when(j + 2 < EK)
        def _():
            pltpu.async_copy(hp_hbm.at[idxs_v.at[j + 2]], rows0, sem)

        pltpu.sync_copy(rows1, acc_sh.at[idxd_v.at[j + 1]], add=True)

    plsc.subcore_barrier()
    rows = NR // NS
    pltpu.sync_copy(acc_sh.at[pl.ds(sid * rows, rows)],
                    out_hbm.at[cid, pl.ds(sid * rows, rows)])


@functools.partial(
    pl.kernel,
    out_type=jax.ShapeDtypeStruct((NC, POOL_R, POOL_W), jnp.float32),
    mesh=_vmesh,
    scratch_types=[
        pltpu.VMEM((PK, PC), jnp.int32),
        pltpu.VMEM((PK, PC, POOL_W), jnp.float32),
        pltpu.VMEM((16, POOL_W), jnp.float32),
        pltpu.VMEM_SHARED((POOL_R, POOL_W), jnp.float32),
    ],
    compiler_params=_sc_params,
)
def _pool_sc(h_hbm, batch_hbm, out_hbm, idx_v, rows_v, zbuf, acc_sh):
    cid = lax.axis_index("c")
    sid = lax.axis_index("s")
    wid = _worker_id()

    _zero_shared_slab(zbuf, acc_sh, 16, POOL_R // (16 * NS))
    plsc.subcore_barrier()

    pltpu.sync_copy(batch_hbm.at[wid], idx_v)
    pltpu.sync_copy(h_hbm.at[wid], rows_v)

    @pl.loop(0, PK)
    def _(j):
        pltpu.sync_copy(rows_v.at[j], acc_sh.at[idx_v.at[j]], add=True)

    plsc.subcore_barrier()
    rows = POOL_R // NS
    pltpu.sync_copy(acc_sh.at[pl.ds(sid * rows, rows)],
                    out_hbm.at[cid, pl.ds(sid * rows, rows)])


def _mm0_body(x_ref, w_ref, o_ref):
    o_ref[...] = jnp.dot(x_ref[...], w_ref[...],
                         preferred_element_type=jnp.float32)


_mm0 = pl.pallas_call(
    _mm0_body,
    out_shape=jax.ShapeDtypeStruct((N_NODES, HID), jnp.float32),
)


def _k1_body(dp_ref, xw_ref, hp_ref, dinv_ref):
    deg = dp_ref[0, 0:N_NODES, 0:1] + dp_ref[1, 0:N_NODES, 0:1] + 1.0
    dinv = lax.rsqrt(deg)
    dinv_ref[...] = dinv
    hp_ref[...] = xw_ref[...] * dinv


_k1 = pl.pallas_call(
    _k1_body,
    out_shape=(
        jax.ShapeDtypeStruct((N_NODES, HID), jnp.float32),
        jax.ShapeDtypeStruct((N_NODES, 1), jnp.float32),
    ),
)


def _gcn_bn_relu(accp_ref, hp_ref, dinv_ref, b_ref, g_ref, be_ref):
    acc = accp_ref[0, 0:N_NODES, :] + accp_ref[1, 0:N_NODES, :] + hp_ref[...]
    gcn = acc * dinv_ref[...] + b_ref[...]
    m = jnp.mean(gcn, axis=0, keepdims=True)
    c = gcn - m
    v = jnp.mean(c * c, axis=0, keepdims=True)
    return jnp.maximum(g_ref[...] * c * lax.rsqrt(v + 1e-5) + be_ref[...], 0.0)


def _post_body(accp_ref, hp_ref, dinv_ref, b_ref, g_ref, be_ref, wn_ref,
               o_ref):
    h = _gcn_bn_relu(accp_ref, hp_ref, dinv_ref, b_ref, g_ref, be_ref)
    o_ref[...] = jnp.dot(h, wn_ref[...],
                         preferred_element_type=jnp.float32) * dinv_ref[...]


_post = pl.pallas_call(
    _post_body,
    out_shape=jax.ShapeDtypeStruct((N_NODES, HID), jnp.float32),
)


def _post3_body(accp_ref, hp_ref, dinv_ref, b_ref, g_ref, be_ref, o_ref):
    h = _gcn_bn_relu(accp_ref, hp_ref, dinv_ref, b_ref, g_ref, be_ref)
    aug = jnp.concatenate([h, jnp.ones((N_NODES, 16), jnp.float32)], axis=1)
    o_ref[pl.ds(0, N_NODES), :] = aug
    o_ref[pl.ds(N_NODES, NR - N_NODES), :] = jnp.zeros(
        (NR - N_NODES, POOL_W), jnp.float32)


_post3 = pl.pallas_call(
    _post3_body,
    out_shape=jax.ShapeDtypeStruct((NR, POOL_W), jnp.float32),
)


def _head_body(pp_ref, w1_ref, b1_ref, w2t_ref, b2_ref, o_ref):
    s = pp_ref[0, 0:NUM_GRAPHS, :] + pp_ref[1, 0:NUM_GRAPHS, :]
    sums = s[:, 0:HID]
    counts = s[:, HID:HID + 1]
    pooled = sums / jnp.maximum(counts, 1.0)
    z = jnp.maximum(
        jnp.dot(pooled, w1_ref[...], preferred_element_type=jnp.float32)
        + b1_ref[...], 0.0)
    o_ref[...] = lax.dot_general(
        w2t_ref[...], z, (((1,), (1,)), ((), ())),
        preferred_element_type=jnp.float32) + b2_ref[...]


_head = pl.pallas_call(
    _head_body,
    out_shape=jax.ShapeDtypeStruct((N_HEADS, NUM_GRAPHS), jnp.float32),
)


def kernel(x, edge_index, edge_attr, batch,
           W1, b1, g1, be1, W2, b2, g2, be2, W3, b3, g3, be3,
           Wh1, bh1, Wh2, bh2):
    del edge_attr

    src = edge_index[0]
    dst = edge_index[1]
    srcp = jnp.pad(src, (0, E_PAD - N_EDGES)).reshape(NW, EK, EC)
    dstp = jnp.pad(dst, (0, E_PAD - N_EDGES),
                   constant_values=GARBAGE_ROW).reshape(NW, EK, EC)
    batchp = jnp.pad(batch, (0, NR - N_NODES),
                     constant_values=NUM_GRAPHS).reshape(NW, PK, PC)

    b1r, g1r, be1r = b1.reshape(1, HID), g1.reshape(1, HID), be1.reshape(1, HID)
    b2r, g2r, be2r = b2.reshape(1, HID), g2.reshape(1, HID), be2.reshape(1, HID)
    b3r, g3r, be3r = b3.reshape(1, HID), g3.reshape(1, HID), be3.reshape(1, HID)
    w1h = Wh1.transpose(1, 0, 2).reshape(HID, N_HEADS * 32)
    b1h = bh1.reshape(1, N_HEADS * 32)
    w2 = Wh2[:, :, 0]
    w2t = (jnp.eye(N_HEADS, dtype=jnp.float32)[:, :, None]
           * w2[:, None, :]).reshape(N_HEADS, N_HEADS * 32)
    b2h = bh2[:, 0:1]

    degp = _deg_sc(dstp)
    xw = _mm0(x, W1)
    h1p, dinv = _k1(degp, xw)

    acc1 = _edge_sc(h1p, srcp, dstp)
    h2p = _post(acc1, h1p, dinv, b1r, g1r, be1r, W2)
    acc2 = _edge_sc(h2p, srcp, dstp)
    h3p = _post(acc2, h2p, dinv, b2r, g2r, be2r, W3)
    acc3 = _edge_sc(h3p, srcp, dstp)
    h3aug = _post3(acc3, h3p, dinv, b3r, g3r, be3r)

    pp = _pool_sc(h3aug.reshape(NW, PK, PC, POOL_W), batchp)
    return _head(pp, w1h, b1h, w2t, b2h)


# speedup vs baseline: 29.7678x; 2.4628x over previous
import functools

import jax
import jax.numpy as jnp
from jax import lax
from jax.experimental import pallas as pl
from jax.experimental.pallas import tpu as pltpu
from jax.experimental.pallas import tpu_sc as plsc

N_NODES = 10000
N_EDGES = 320000
IN_DIM = 128
HID = 64
NUM_GRAPHS = 128
N_HEADS = 5

NC = 2
NS = 16
NW = NC * NS
EC = 128
EK = 80
E_PAD = NW * EK * EC
NR = 10240
GARBAGE_ROW = 10200

PK = 5
PC = 64
POOL_W = HID + 16
POOL_R = 256

_vmesh = plsc.VectorSubcoreMesh(core_axis_name="c", subcore_axis_name="s")
_sc_params = pltpu.CompilerParams(use_tc_tiling_on_sc=False)


def _worker_id():
    return lax.axis_index("c") * NS + lax.axis_index("s")


def _zero_shared_slab(zbuf, acc_sh, rows_per_copy, copies_per_sub):
    sid = lax.axis_index("s")
    width = zbuf.shape[-1]

    @pl.loop(0, zbuf.shape[0])
    def _(i):
        @pl.loop(0, width // 16)
        def _(c):
            zbuf[i, pl.ds(c * 16, 16)] = jnp.zeros((16,), jnp.float32)

    base = sid * rows_per_copy * copies_per_sub

    @pl.loop(0, copies_per_sub)
    def _(t):
        pltpu.sync_copy(zbuf, acc_sh.at[pl.ds(base + t * rows_per_copy,
                                              rows_per_copy)])


@functools.partial(
    pl.kernel,
    out_type=jax.ShapeDtypeStruct((NC, NR, 16), jnp.float32),
    mesh=_vmesh,
    scratch_types=[
        pltpu.VMEM((EK, EC), jnp.int32),
        pltpu.VMEM((EC, 16), jnp.float32),
        pltpu.VMEM((128, 16), jnp.float32),
        pltpu.VMEM_SHARED((NR, 16), jnp.float32),
    ],
    compiler_params=_sc_params,
)
def _deg_sc(dst_hbm, out_hbm, idx_v, ones_v, zbuf, acc_sh):
    cid = lax.axis_index("c")
    sid = lax.axis_index("s")
    wid = _worker_id()

    @pl.loop(0, EC)
    def _(i):
        ones_v[i, :] = jnp.ones((16,), jnp.float32)

    _zero_shared_slab(zbuf, acc_sh, 128, NR // (128 * NS))
    plsc.subcore_barrier()

    pltpu.sync_copy(dst_hbm.at[wid], idx_v)

    @pl.loop(0, EK)
    def _(j):
        pltpu.sync_copy(ones_v, acc_sh.at[idx_v.at[j]], add=True)

    plsc.subcore_barrier()
    rows = NR // NS
    pltpu.sync_copy(acc_sh.at[pl.ds(sid * rows, rows)],
                    out_hbm.at[cid, pl.ds(sid * rows, rows)])


@functools.partial(
    pl.kernel,
    out_type=jax.ShapeDtypeStruct((NC, NR, HID), jnp.float32),
    mesh=_vmesh,
    scratch_types=[
        pltpu.VMEM((EK, EC), jnp.int32),
        pltpu.VMEM((EK, EC), jnp.int32),
        pltpu.VMEM((EC, HID), jnp.float32),
        pltpu.VMEM((EC, HID), jnp.float32),
        pltpu.VMEM((128, HID), jnp.float32),
        pltpu.VMEM_SHARED((NR, HID), jnp.float32),
        pltpu.VMEM_SHARED((N_NODES, HID), jnp.float32),
        pltpu.SemaphoreType.DMA,
    ],
    compiler_params=_sc_params,
)
def _edge_sc(hp_hbm, src_hbm, dst_hbm, out_hbm,
             idxs_v, idxd_v, rows0, rows1, zbuf, acc_sh, hp_sh, sem):
    cid = lax.axis_index("c")
    sid = lax.axis_index("s")
    wid = _worker_id()

    _zero_shared_slab(zbuf, acc_sh, 128, NR // (128 * NS))
    hrows = N_NODES // NS
    pltpu.sync_copy(hp_hbm.at[pl.ds(sid * hrows, hrows)],
                    hp_sh.at[pl.ds(sid * hrows, hrows)])
    plsc.subcore_barrier()

    pltpu.sync_copy(src_hbm.at[wid], idxs_v)
    pltpu.sync_copy(dst_hbm.at[wid], idxd_v)

    pltpu.async_copy(hp_sh.at[idxs_v.at[0]], rows0, sem)

    @pl.loop(0, EK, step=2)
    def _(j):
        pltpu.make_async_copy(hp_sh.at[idxs_v.at[j]], rows0, sem).wait()
        pltpu.async_copy(hp_sh.at[idxs_v.at[j + 1]], rows1, sem)
        pltpu.sync_copy(rows0, acc_sh.at[idxd_v.at[j]], add=True)

        pltpu.make_async_copy(hp_sh.at[idxs_v.at[j + 1]], rows1, sem).wait()

        @pl.when(j + 2 < EK)
        def _():
            pltpu.async_copy(hp_sh.at[idxs_v.at[j + 2]], rows0, sem)

        pltpu.sync_copy(rows1, acc_sh.at[idxd_v.at[j + 1]], add=True)

    plsc.subcore_barrier()
    rows = NR // NS
    pltpu.sync_copy(acc_sh.at[pl.ds(sid * rows, rows)],
                    out_hbm.at[cid, pl.ds(sid * rows, rows)])


@functools.partial(
    pl.kernel,
    out_type=jax.ShapeDtypeStruct((NC, POOL_R, POOL_W), jnp.float32),
    mesh=_vmesh,
    scratch_types=[
        pltpu.VMEM((PK, PC), jnp.int32),
        pltpu.VMEM((PK, PC, POOL_W), jnp.float32),
        pltpu.VMEM((16, POOL_W), jnp.float32),
        pltpu.VMEM_SHARED((POOL_R, POOL_W), jnp.float32),
    ],
    compiler_params=_sc_params,
)
def _pool_sc(h_hbm, batch_hbm, out_hbm, idx_v, rows_v, zbuf, acc_sh):
    cid = lax.axis_index("c")
    sid = lax.axis_index("s")
    wid = _worker_id()

    _zero_shared_slab(zbuf, acc_sh, 16, POOL_R // (16 * NS))
    plsc.subcore_barrier()

    pltpu.sync_copy(batch_hbm.at[wid], idx_v)
    pltpu.sync_copy(h_hbm.at[wid], rows_v)

    @pl.loop(0, PK)
    def _(j):
        pltpu.sync_copy(rows_v.at[j], acc_sh.at[idx_v.at[j]], add=True)

    plsc.subcore_barrier()
    rows = POOL_R // NS
    pltpu.sync_copy(acc_sh.at[pl.ds(sid * rows, rows)],
                    out_hbm.at[cid, pl.ds(sid * rows, rows)])


def _mm0_body(x_ref, w_ref, o_ref):
    o_ref[...] = jnp.dot(x_ref[...], w_ref[...],
                         preferred_element_type=jnp.float32)


_mm0 = pl.pallas_call(
    _mm0_body,
    out_shape=jax.ShapeDtypeStruct((N_NODES, HID), jnp.float32),
)


def _k1_body(dp_ref, xw_ref, hp_ref, dinv_ref):
    deg = dp_ref[0, 0:N_NODES, 0:1] + dp_ref[1, 0:N_NODES, 0:1] + 1.0
    dinv = lax.rsqrt(deg)
    dinv_ref[...] = dinv
    hp_ref[...] = xw_ref[...] * dinv


_k1 = pl.pallas_call(
    _k1_body,
    out_shape=(
        jax.ShapeDtypeStruct((N_NODES, HID), jnp.float32),
        jax.ShapeDtypeStruct((N_NODES, 1), jnp.float32),
    ),
)


def _gcn_bn_relu(accp_ref, hp_ref, dinv_ref, b_ref, g_ref, be_ref):
    acc = accp_ref[0, 0:N_NODES, :] + accp_ref[1, 0:N_NODES, :] + hp_ref[...]
    gcn = acc * dinv_ref[...] + b_ref[...]
    m = jnp.mean(gcn, axis=0, keepdims=True)
    c = gcn - m
    v = jnp.mean(c * c, axis=0, keepdims=True)
    return jnp.maximum(g_ref[...] * c * lax.rsqrt(v + 1e-5) + be_ref[...], 0.0)


def _post_body(accp_ref, hp_ref, dinv_ref, b_ref, g_ref, be_ref, wn_ref,
               o_ref):
    h = _gcn_bn_relu(accp_ref, hp_ref, dinv_ref, b_ref, g_ref, be_ref)
    o_ref[...] = jnp.dot(h, wn_ref[...],
                         preferred_element_type=jnp.float32) * dinv_ref[...]


_post = pl.pallas_call(
    _post_body,
    out_shape=jax.ShapeDtypeStruct((N_NODES, HID), jnp.float32),
)


def _post3_body(accp_ref, hp_ref, dinv_ref, b_ref, g_ref, be_ref, o_ref):
    h = _gcn_bn_relu(accp_ref, hp_ref, dinv_ref, b_ref, g_ref, be_ref)
    aug = jnp.concatenate([h, jnp.ones((N_NODES, 16), jnp.float32)], axis=1)
    o_ref[pl.ds(0, N_NODES), :] = aug
    o_ref[pl.ds(N_NODES, NR - N_NODES), :] = jnp.zeros(
        (NR - N_NODES, POOL_W), jnp.float32)


_post3 = pl.pallas_call(
    _post3_body,
    out_shape=jax.ShapeDtypeStruct((NR, POOL_W), jnp.float32),
)


def _head_body(pp_ref, w1_ref, b1_ref, w2t_ref, b2_ref, o_ref):
    s = pp_ref[0, 0:NUM_GRAPHS, :] + pp_ref[1, 0:NUM_GRAPHS, :]
    sums = s[:, 0:HID]
    counts = s[:, HID:HID + 1]
    pooled = sums / jnp.maximum(counts, 1.0)
    z = jnp.maximum(
        jnp.dot(pooled, w1_ref[...], preferred_element_type=jnp.float32)
        + b1_ref[...], 0.0)
    o_ref[...] = lax.dot_general(
        w2t_ref[...], z, (((1,), (1,)), ((), ())),
        preferred_element_type=jnp.float32) + b2_ref[...]


_head = pl.pallas_call(
    _head_body,
    out_shape=jax.ShapeDtypeStruct((N_HEADS, NUM_GRAPHS), jnp.float32),
)


def kernel(x, edge_index, edge_attr, batch,
           W1, b1, g1, be1, W2, b2, g2, be2, W3, b3, g3, be3,
           Wh1, bh1, Wh2, bh2):
    del edge_attr

    src = edge_index[0]
    dst = edge_index[1]
    srcp = jnp.pad(src, (0, E_PAD - N_EDGES)).reshape(NW, EK, EC)
    dstp = jnp.pad(dst, (0, E_PAD - N_EDGES),
                   constant_values=GARBAGE_ROW).reshape(NW, EK, EC)
    batchp = jnp.pad(batch, (0, NR - N_NODES),
                     constant_values=NUM_GRAPHS).reshape(NW, PK, PC)

    b1r, g1r, be1r = b1.reshape(1, HID), g1.reshape(1, HID), be1.reshape(1, HID)
    b2r, g2r, be2r = b2.reshape(1, HID), g2.reshape(1, HID), be2.reshape(1, HID)
    b3r, g3r, be3r = b3.reshape(1, HID), g3.reshape(1, HID), be3.reshape(1, HID)
    w1h = Wh1.transpose(1, 0, 2).reshape(HID, N_HEADS * 32)
    b1h = bh1.reshape(1, N_HEADS * 32)
    w2 = Wh2[:, :, 0]
    w2t = (jnp.eye(N_HEADS, dtype=jnp.float32)[:, :, None]
           * w2[:, None, :]).reshape(N_HEADS, N_HEADS * 32)
    b2h = bh2[:, 0:1]

    degp = _deg_sc(dstp)
    xw = _mm0(x, W1)
    h1p, dinv = _k1(degp, xw)

    acc1 = _edge_sc(h1p, srcp, dstp)
    h2p = _post(acc1, h1p, dinv, b1r, g1r, be1r, W2)
    acc2 = _edge_sc(h2p, srcp, dstp)
    h3p = _post(acc2, h2p, dinv, b2r, g2r, be2r, W3)
    acc3 = _edge_sc(h3p, srcp, dstp)
    h3aug = _post3(acc3, h3p, dinv, b3r, g3r, be3r)

    pp = _pool_sc(h3aug.reshape(NW, PK, PC, POOL_W), batchp)
    return _head(pp, w1h, b1h, w2t, b2h)


# speedup vs baseline: 29.8161x; 1.0016x over previous
import functools

import jax
import jax.numpy as jnp
from jax import lax
from jax.experimental import pallas as pl
from jax.experimental.pallas import tpu as pltpu
from jax.experimental.pallas import tpu_sc as plsc

N_NODES = 10000
N_EDGES = 320000
IN_DIM = 128
HID = 64
NUM_GRAPHS = 128
N_HEADS = 5

NC = 2
NS = 16
NW = NC * NS
EC = 128
EK = 80
E_PAD = NW * EK * EC
NR = 10240
GARBAGE_ROW = 10200

PK = 5
PC = 64
POOL_W = HID + 16
POOL_R = 256

_vmesh = plsc.VectorSubcoreMesh(core_axis_name="c", subcore_axis_name="s")
_sc_params = pltpu.CompilerParams(use_tc_tiling_on_sc=False)


def _worker_id():
    return lax.axis_index("c") * NS + lax.axis_index("s")


def _zero_shared_slab(zbuf, acc_sh, rows_per_copy, copies_per_sub):
    sid = lax.axis_index("s")
    width = zbuf.shape[-1]

    @pl.loop(0, zbuf.shape[0])
    def _(i):
        @pl.loop(0, width // 16)
        def _(c):
            zbuf[i, pl.ds(c * 16, 16)] = jnp.zeros((16,), jnp.float32)

    base = sid * rows_per_copy * copies_per_sub

    @pl.loop(0, copies_per_sub)
    def _(t):
        pltpu.sync_copy(zbuf, acc_sh.at[pl.ds(base + t * rows_per_copy,
                                              rows_per_copy)])


@functools.partial(
    pl.kernel,
    out_type=jax.ShapeDtypeStruct((NC, NR, 16), jnp.float32),
    mesh=_vmesh,
    scratch_types=[
        pltpu.VMEM((EK, EC), jnp.int32),
        pltpu.VMEM((EC, 16), jnp.float32),
        pltpu.VMEM((128, 16), jnp.float32),
        pltpu.VMEM_SHARED((NR, 16), jnp.float32),
    ],
    compiler_params=_sc_params,
)
def _deg_sc(dst_hbm, out_hbm, idx_v, ones_v, zbuf, acc_sh):
    cid = lax.axis_index("c")
    sid = lax.axis_index("s")
    wid = _worker_id()

    @pl.loop(0, EC)
    def _(i):
        ones_v[i, :] = jnp.ones((16,), jnp.float32)

    _zero_shared_slab(zbuf, acc_sh, 128, NR // (128 * NS))
    plsc.subcore_barrier()

    pltpu.sync_copy(dst_hbm.at[wid], idx_v)

    @pl.loop(0, EK)
    def _(j):
        pltpu.sync_copy(ones_v, acc_sh.at[idx_v.at[j]], add=True)

    plsc.subcore_barrier()
    rows = NR // NS
    pltpu.sync_copy(acc_sh.at[pl.ds(sid * rows, rows)],
                    out_hbm.at[cid, pl.ds(sid * rows, rows)])


@functools.partial(
    pl.kernel,
    out_type=jax.ShapeDtypeStruct((NC, NR, HID), jnp.float32),
    mesh=_vmesh,
    scratch_types=[
        pltpu.VMEM((EK, EC), jnp.int32),
        pltpu.VMEM((EK, EC), jnp.int32),
        pltpu.VMEM((EC, HID), jnp.float32),
        pltpu.VMEM((EC, HID), jnp.float32),
        pltpu.VMEM((EC, HID), jnp.float32),
        pltpu.VMEM((EC, HID), jnp.float32),
        pltpu.VMEM((128, HID), jnp.float32),
        pltpu.VMEM_SHARED((NR, HID), jnp.float32),
        pltpu.VMEM_SHARED((N_NODES, HID), jnp.float32),
        pltpu.SemaphoreType.DMA,
    ],
    compiler_params=_sc_params,
)
def _edge_sc(hp_hbm, src_hbm, dst_hbm, out_hbm,
             idxs_v, idxd_v, rows0, rows1, rows2, rows3, zbuf, acc_sh, hp_sh,
             sem):
    cid = lax.axis_index("c")
    sid = lax.axis_index("s")
    wid = _worker_id()

    _zero_shared_slab(zbuf, acc_sh, 128, NR // (128 * NS))
    hrows = N_NODES // NS
    pltpu.sync_copy(hp_hbm.at[pl.ds(sid * hrows, hrows)],
                    hp_sh.at[pl.ds(sid * hrows, hrows)])
    plsc.subcore_barrier()

    pltpu.sync_copy(src_hbm.at[wid], idxs_v)
    pltpu.sync_copy(dst_hbm.at[wid], idxd_v)

    pltpu.async_copy(hp_sh.at[idxs_v.at[0]], rows0, sem)

    @pl.loop(0, EK, step=2)
    def _(j):
        pltpu.make_async_copy(hp_sh.at[idxs_v.at[j]], rows0, sem).wait()
        pltpu.async_copy(hp_sh.at[idxs_v.at[j + 1]], rows1, sem)
        pltpu.sync_copy(rows0, acc_sh.at[idxd_v.at[j]], add=True)

        pltpu.make_async_copy(hp_sh.at[idxs_v.at[j + 1]], rows1, sem).wait()

        @pl.when(j + 2 < EK)
        def _():
            pltpu.async_copy(hp_sh.at[idxs_v.at[j + 2]], rows0, sem)

        pltpu.sync_copy(rows1, acc_sh.at[idxd_v.at[j + 1]], add=True)

    plsc.subcore_barrier()
    rows = NR // NS
    pltpu.sync_copy(acc_sh.at[pl.ds(sid * rows, rows)],
                    out_hbm.at[cid, pl.ds(sid * rows, rows)])


@functools.partial(
    pl.kernel,
    out_type=jax.ShapeDtypeStruct((NC, POOL_R, POOL_W), jnp.float32),
    mesh=_vmesh,
    scratch_types=[
        pltpu.VMEM((PK, PC), jnp.int32),
        pltpu.VMEM((PK, PC, POOL_W), jnp.float32),
        pltpu.VMEM((16, POOL_W), jnp.float32),
        pltpu.VMEM_SHARED((POOL_R, POOL_W), jnp.float32),
    ],
    compiler_params=_sc_params,
)
def _pool_sc(h_hbm, batch_hbm, out_hbm, idx_v, rows_v, zbuf, acc_sh):
    cid = lax.axis_index("c")
    sid = lax.axis_index("s")
    wid = _worker_id()

    _zero_shared_slab(zbuf, acc_sh, 16, POOL_R // (16 * NS))
    plsc.subcore_barrier()

    pltpu.sync_copy(batch_hbm.at[wid], idx_v)
    pltpu.sync_copy(h_hbm.at[wid], rows_v)

    @pl.loop(0, PK)
    def _(j):
        pltpu.sync_copy(rows_v.at[j], acc_sh.at[idx_v.at[j]], add=True)

    plsc.subcore_barrier()
    rows = POOL_R // NS
    pltpu.sync_copy(acc_sh.at[pl.ds(sid * rows, rows)],
                    out_hbm.at[cid, pl.ds(sid * rows, rows)])


def _k1_body(dp_ref, x_ref, w_ref, hp_ref, dinv_ref):
    deg = dp_ref[0, 0:N_NODES, 0:1] + dp_ref[1, 0:N_NODES, 0:1] + 1.0
    dinv = lax.rsqrt(deg)
    dinv_ref[...] = dinv
    hp_ref[...] = jnp.dot(x_ref[...], w_ref[...],
                          preferred_element_type=jnp.float32) * dinv


_k1 = pl.pallas_call(
    _k1_body,
    out_shape=(
        jax.ShapeDtypeStruct((N_NODES, HID), jnp.float32),
        jax.ShapeDtypeStruct((N_NODES, 1), jnp.float32),
    ),
)


def _gcn_bn_relu(accp_ref, hp_ref, dinv_ref, b_ref, g_ref, be_ref):
    acc = accp_ref[0, 0:N_NODES, :] + accp_ref[1, 0:N_NODES, :] + hp_ref[...]
    gcn = acc * dinv_ref[...] + b_ref[...]
    m = jnp.mean(gcn, axis=0, keepdims=True)
    c = gcn - m
    v = jnp.mean(c * c, axis=0, keepdims=True)
    return jnp.maximum(g_ref[...] * c * lax.rsqrt(v + 1e-5) + be_ref[...], 0.0)


def _post_body(accp_ref, hp_ref, dinv_ref, b_ref, g_ref, be_ref, wn_ref,
               o_ref):
    h = _gcn_bn_relu(accp_ref, hp_ref, dinv_ref, b_ref, g_ref, be_ref)
    o_ref[...] = jnp.dot(h, wn_ref[...],
                         preferred_element_type=jnp.float32) * dinv_ref[...]


_post = pl.pallas_call(
    _post_body,
    out_shape=jax.ShapeDtypeStruct((N_NODES, HID), jnp.float32),
)


def _post3_body(accp_ref, hp_ref, dinv_ref, b_ref, g_ref, be_ref, o_ref):
    h = _gcn_bn_relu(accp_ref, hp_ref, dinv_ref, b_ref, g_ref, be_ref)
    aug = jnp.concatenate([h, jnp.ones((N_NODES, 16), jnp.float32)], axis=1)
    o_ref[pl.ds(0, N_NODES), :] = aug
    o_ref[pl.ds(N_NODES, NR - N_NODES), :] = jnp.zeros(
        (NR - N_NODES, POOL_W), jnp.float32)


_post3 = pl.pallas_call(
    _post3_body,
    out_shape=jax.ShapeDtypeStruct((NR, POOL_W), jnp.float32),
)


def _head_body(pp_ref, w1_ref, b1_ref, w2t_ref, b2_ref, o_ref):
    s = pp_ref[0, 0:NUM_GRAPHS, :] + pp_ref[1, 0:NUM_GRAPHS, :]
    sums = s[:, 0:HID]
    counts = s[:, HID:HID + 1]
    pooled = sums / jnp.maximum(counts, 1.0)
    z = jnp.maximum(
        jnp.dot(pooled, w1_ref[...], preferred_element_type=jnp.float32)
        + b1_ref[...], 0.0)
    o_ref[...] = lax.dot_general(
        w2t_ref[...], z, (((1,), (1,)), ((), ())),
        preferred_element_type=jnp.float32) + b2_ref[...]


_head = pl.pallas_call(
    _head_body,
    out_shape=jax.ShapeDtypeStruct((N_HEADS, NUM_GRAPHS), jnp.float32),
)


def kernel(x, edge_index, edge_attr, batch,
           W1, b1, g1, be1, W2, b2, g2, be2, W3, b3, g3, be3,
           Wh1, bh1, Wh2, bh2):
    del edge_attr

    src = edge_index[0]
    dst = edge_index[1]
    srcp = jnp.pad(src, (0, E_PAD - N_EDGES)).reshape(NW, EK, EC)
    dstp = jnp.pad(dst, (0, E_PAD - N_EDGES),
                   constant_values=GARBAGE_ROW).reshape(NW, EK, EC)
    batchp = jnp.pad(batch, (0, NR - N_NODES),
                     constant_values=NUM_GRAPHS).reshape(NW, PK, PC)

    b1r, g1r, be1r = b1.reshape(1, HID), g1.reshape(1, HID), be1.reshape(1, HID)
    b2r, g2r, be2r = b2.reshape(1, HID), g2.reshape(1, HID), be2.reshape(1, HID)
    b3r, g3r, be3r = b3.reshape(1, HID), g3.reshape(1, HID), be3.reshape(1, HID)
    w1h = Wh1.transpose(1, 0, 2).reshape(HID, N_HEADS * 32)
    b1h = bh1.reshape(1, N_HEADS * 32)
    w2 = Wh2[:, :, 0]
    w2t = (jnp.eye(N_HEADS, dtype=jnp.float32)[:, :, None]
           * w2[:, None, :]).reshape(N_HEADS, N_HEADS * 32)
    b2h = bh2[:, 0:1]

    degp = _deg_sc(dstp)
    h1p, dinv = _k1(degp, x, W1)

    acc1 = _edge_sc(h1p, srcp, dstp)
    h2p = _post(acc1, h1p, dinv, b1r, g1r, be1r, W2)
    acc2 = _edge_sc(h2p, srcp, dstp)
    h3p = _post(acc2, h2p, dinv, b2r, g2r, be2r, W3)
    acc3 = _edge_sc(h3p, srcp, dstp)
    h3aug = _post3(acc3, h3p, dinv, b3r, g3r, be3r)

    pp = _pool_sc(h3aug.reshape(NW, PK, PC, POOL_W), batchp)
    return _head(pp, w1h, b1h, w2t, b2h)


# speedup vs baseline: 30.7809x; 1.0324x over previous
import functools

import jax
import jax.numpy as jnp
from jax import lax
from jax.experimental import pallas as pl
from jax.experimental.pallas import tpu as pltpu
from jax.experimental.pallas import tpu_sc as plsc

N_NODES = 10000
N_EDGES = 320000
IN_DIM = 128
HID = 64
NUM_GRAPHS = 128
N_HEADS = 5

NC = 2
NS = 16
NW = NC * NS
EC = 128
EK = 80
E_PAD = NW * EK * EC
NR = 10240
GARBAGE_ROW = 10200

PK = 5
PC = 64
POOL_W = HID + 16
POOL_R = 256

_vmesh = plsc.VectorSubcoreMesh(core_axis_name="c", subcore_axis_name="s")
_sc_params = pltpu.CompilerParams(use_tc_tiling_on_sc=False)


def _worker_id():
    return lax.axis_index("c") * NS + lax.axis_index("s")


def _zero_shared_slab(zbuf, acc_sh, rows_per_copy, copies_per_sub):
    sid = lax.axis_index("s")
    width = zbuf.shape[-1]

    @pl.loop(0, zbuf.shape[0])
    def _(i):
        @pl.loop(0, width // 16)
        def _(c):
            zbuf[i, pl.ds(c * 16, 16)] = jnp.zeros((16,), jnp.float32)

    base = sid * rows_per_copy * copies_per_sub

    @pl.loop(0, copies_per_sub)
    def _(t):
        pltpu.sync_copy(zbuf, acc_sh.at[pl.ds(base + t * rows_per_copy,
                                              rows_per_copy)])


@functools.partial(
    pl.kernel,
    out_type=jax.ShapeDtypeStruct((NC, NR, 16), jnp.float32),
    mesh=_vmesh,
    scratch_types=[
        pltpu.VMEM((EK, EC), jnp.int32),
        pltpu.VMEM((EC, 16), jnp.float32),
        pltpu.VMEM((128, 16), jnp.float32),
        pltpu.VMEM_SHARED((NR, 16), jnp.float32),
    ],
    compiler_params=_sc_params,
)
def _deg_sc(dst_hbm, out_hbm, idx_v, ones_v, zbuf, acc_sh):
    cid = lax.axis_index("c")
    sid = lax.axis_index("s")
    wid = _worker_id()

    @pl.loop(0, EC)
    def _(i):
        ones_v[i, :] = jnp.ones((16,), jnp.float32)

    _zero_shared_slab(zbuf, acc_sh, 128, NR // (128 * NS))
    plsc.subcore_barrier()

    pltpu.sync_copy(dst_hbm.at[wid], idx_v)

    @pl.loop(0, EK)
    def _(j):
        pltpu.sync_copy(ones_v, acc_sh.at[idx_v.at[j]], add=True)

    plsc.subcore_barrier()
    rows = NR // NS
    pltpu.sync_copy(acc_sh.at[pl.ds(sid * rows, rows)],
                    out_hbm.at[cid, pl.ds(sid * rows, rows)])


@functools.partial(
    pl.kernel,
    out_type=jax.ShapeDtypeStruct((NC, NR, HID), jnp.float32),
    mesh=_vmesh,
    scratch_types=[
        pltpu.VMEM((EK, EC), jnp.int32),
        pltpu.VMEM((EK, EC), jnp.int32),
        pltpu.VMEM((EC, HID), jnp.float32),
        pltpu.VMEM((EC, HID), jnp.float32),
        pltpu.VMEM((EC, HID), jnp.float32),
        pltpu.VMEM((EC, HID), jnp.float32),
        pltpu.VMEM((128, HID), jnp.float32),
        pltpu.VMEM_SHARED((NR, HID), jnp.float32),
        pltpu.VMEM_SHARED((N_NODES, HID), jnp.float32),
        pltpu.SemaphoreType.DMA,
    ],
    compiler_params=_sc_params,
)
def _edge_sc(hp_hbm, src_hbm, dst_hbm, out_hbm,
             idxs_v, idxd_v, rows0, rows1, rows2, rows3, zbuf, acc_sh, hp_sh,
             sem):
    cid = lax.axis_index("c")
    sid = lax.axis_index("s")
    wid = _worker_id()

    _zero_shared_slab(zbuf, acc_sh, 128, NR // (128 * NS))
    hrows = N_NODES // NS
    pltpu.sync_copy(hp_hbm.at[pl.ds(sid * hrows, hrows)],
                    hp_sh.at[pl.ds(sid * hrows, hrows)])
    plsc.subcore_barrier()

    pltpu.sync_copy(src_hbm.at[wid], idxs_v)
    pltpu.sync_copy(dst_hbm.at[wid], idxd_v)

    pltpu.async_copy(hp_sh.at[idxs_v.at[0]], rows0, sem)

    @pl.loop(0, EK, step=2)
    def _(j):
        pltpu.make_async_copy(hp_sh.at[idxs_v.at[j]], rows0, sem).wait()
        pltpu.async_copy(hp_sh.at[idxs_v.at[j + 1]], rows1, sem)
        pltpu.sync_copy(rows0, acc_sh.at[idxd_v.at[j]], add=True)

        pltpu.make_async_copy(hp_sh.at[idxs_v.at[j + 1]], rows1, sem).wait()

        @pl.when(j + 2 < EK)
        def _():
            pltpu.async_copy(hp_sh.at[idxs_v.at[j + 2]], rows0, sem)

        pltpu.sync_copy(rows1, acc_sh.at[idxd_v.at[j + 1]], add=True)

    plsc.subcore_barrier()
    rows = NR // NS
    pltpu.sync_copy(acc_sh.at[pl.ds(sid * rows, rows)],
                    out_hbm.at[cid, pl.ds(sid * rows, rows)])


def _k1_body(dp_ref, x_ref, w_ref, hp_ref, dinv_ref):
    deg = dp_ref[0, 0:N_NODES, 0:1] + dp_ref[1, 0:N_NODES, 0:1] + 1.0
    dinv = lax.rsqrt(deg)
    dinv_ref[...] = dinv
    hp_ref[...] = jnp.dot(x_ref[...], w_ref[...],
                          preferred_element_type=jnp.float32) * dinv


_k1 = pl.pallas_call(
    _k1_body,
    out_shape=(
        jax.ShapeDtypeStruct((N_NODES, HID), jnp.float32),
        jax.ShapeDtypeStruct((N_NODES, 1), jnp.float32),
    ),
)


def _gcn_bn_relu(accp_ref, hp_ref, dinv_ref, b_ref, g_ref, be_ref):
    acc = accp_ref[0, 0:N_NODES, :] + accp_ref[1, 0:N_NODES, :] + hp_ref[...]
    gcn = acc * dinv_ref[...] + b_ref[...]
    m = jnp.mean(gcn, axis=0, keepdims=True)
    c = gcn - m
    v = jnp.mean(c * c, axis=0, keepdims=True)
    return jnp.maximum(g_ref[...] * c * lax.rsqrt(v + 1e-5) + be_ref[...], 0.0)


def _post_body(accp_ref, hp_ref, dinv_ref, b_ref, g_ref, be_ref, wn_ref,
               o_ref):
    h = _gcn_bn_relu(accp_ref, hp_ref, dinv_ref, b_ref, g_ref, be_ref)
    o_ref[...] = jnp.dot(h, wn_ref[...],
                         preferred_element_type=jnp.float32) * dinv_ref[...]


_post = pl.pallas_call(
    _post_body,
    out_shape=jax.ShapeDtypeStruct((N_NODES, HID), jnp.float32),
)


def _post3_body(accp_ref, hp_ref, dinv_ref, b_ref, g_ref, be_ref, o_ref):
    h = _gcn_bn_relu(accp_ref, hp_ref, dinv_ref, b_ref, g_ref, be_ref)
    o_ref[...] = jnp.concatenate(
        [h, jnp.ones((N_NODES, 16), jnp.float32)], axis=1)


_post3 = pl.pallas_call(
    _post3_body,
    out_shape=jax.ShapeDtypeStruct((N_NODES, POOL_W), jnp.float32),
)


def _head_body(h_ref, batch_ref, w1_ref, b1_ref, w2t_ref, b2_ref, o_ref):
    gids = lax.broadcasted_iota(jnp.int32, (NUM_GRAPHS, N_NODES), 0)
    mask = (batch_ref[...] == gids).astype(jnp.float32)
    s = jnp.dot(mask, h_ref[...], preferred_element_type=jnp.float32)
    sums = s[:, 0:HID]
    counts = s[:, HID:HID + 1]
    pooled = sums / jnp.maximum(counts, 1.0)
    z = jnp.maximum(
        jnp.dot(pooled, w1_ref[...], preferred_element_type=jnp.float32)
        + b1_ref[...], 0.0)
    o_ref[...] = lax.dot_general(
        w2t_ref[...], z, (((1,), (1,)), ((), ())),
        preferred_element_type=jnp.float32) + b2_ref[...]


_head = pl.pallas_call(
    _head_body,
    out_shape=jax.ShapeDtypeStruct((N_HEADS, NUM_GRAPHS), jnp.float32),
)


def kernel(x, edge_index, edge_attr, batch,
           W1, b1, g1, be1, W2, b2, g2, be2, W3, b3, g3, be3,
           Wh1, bh1, Wh2, bh2):
    del edge_attr

    src = edge_index[0]
    dst = edge_index[1]
    srcp = jnp.pad(src, (0, E_PAD - N_EDGES)).reshape(NW, EK, EC)
    dstp = jnp.pad(dst, (0, E_PAD - N_EDGES),
                   constant_values=GARBAGE_ROW).reshape(NW, EK, EC)

    b1r, g1r, be1r = b1.reshape(1, HID), g1.reshape(1, HID), be1.reshape(1, HID)
    b2r, g2r, be2r = b2.reshape(1, HID), g2.reshape(1, HID), be2.reshape(1, HID)
    b3r, g3r, be3r = b3.reshape(1, HID), g3.reshape(1, HID), be3.reshape(1, HID)
    w1h = Wh1.transpose(1, 0, 2).reshape(HID, N_HEADS * 32)
    b1h = bh1.reshape(1, N_HEADS * 32)
    w2 = Wh2[:, :, 0]
    w2t = (jnp.eye(N_HEADS, dtype=jnp.float32)[:, :, None]
           * w2[:, None, :]).reshape(N_HEADS, N_HEADS * 32)
    b2h = bh2[:, 0:1]

    degp = _deg_sc(dstp)
    h1p, dinv = _k1(degp, x, W1)

    acc1 = _edge_sc(h1p, srcp, dstp)
    h2p = _post(acc1, h1p, dinv, b1r, g1r, be1r, W2)
    acc2 = _edge_sc(h2p, srcp, dstp)
    h3p = _post(acc2, h2p, dinv, b2r, g2r, be2r, W3)
    acc3 = _edge_sc(h3p, srcp, dstp)
    h3aug = _post3(acc3, h3p, dinv, b3r, g3r, be3r)

    return _head(h3aug, batch.reshape(1, N_NODES), w1h, b1h, w2t, b2h)


# speedup vs baseline: 41.7311x; 1.3557x over previous
import functools

import jax
import jax.numpy as jnp
from jax import lax
from jax.experimental import pallas as pl
from jax.experimental.pallas import tpu as pltpu
from jax.experimental.pallas import tpu_sc as plsc

N_NODES = 10000
N_EDGES = 320000
IN_DIM = 128
HID = 64
NUM_GRAPHS = 128
N_HEADS = 5

NC = 2
NS = 16
NW = NC * NS
EC = 128
EK = 80
E_PAD = NW * EK * EC
NR = 10240
GARBAGE_ROW = 10200

PK = 5
PC = 64
POOL_W = HID + 16
POOL_R = 256

_vmesh = plsc.VectorSubcoreMesh(core_axis_name="c", subcore_axis_name="s")
_sc_params = pltpu.CompilerParams(use_tc_tiling_on_sc=False)


def _worker_id():
    return lax.axis_index("c") * NS + lax.axis_index("s")


def _zero_shared_slab(zbuf, acc_sh, rows_per_copy, copies_per_sub):
    sid = lax.axis_index("s")
    width = zbuf.shape[-1]
    vw = 32 if zbuf.dtype == jnp.bfloat16 else 16

    @pl.loop(0, zbuf.shape[0])
    def _(i):
        @pl.loop(0, width // vw)
        def _(c):
            zbuf[i, pl.ds(c * vw, vw)] = jnp.zeros((vw,), zbuf.dtype)

    base = sid * rows_per_copy * copies_per_sub

    @pl.loop(0, copies_per_sub)
    def _(t):
        pltpu.sync_copy(zbuf, acc_sh.at[pl.ds(base + t * rows_per_copy,
                                              rows_per_copy)])


@functools.partial(
    pl.kernel,
    out_type=jax.ShapeDtypeStruct((NC, NR, 16), jnp.float32),
    mesh=_vmesh,
    scratch_types=[
        pltpu.VMEM((EK, EC), jnp.int32),
        pltpu.VMEM((EC, 16), jnp.float32),
        pltpu.VMEM((128, 16), jnp.float32),
        pltpu.VMEM_SHARED((NR, 16), jnp.float32),
    ],
    compiler_params=_sc_params,
)
def _deg_sc(dst_hbm, out_hbm, idx_v, ones_v, zbuf, acc_sh):
    cid = lax.axis_index("c")
    sid = lax.axis_index("s")
    wid = _worker_id()

    @pl.loop(0, EC)
    def _(i):
        ones_v[i, :] = jnp.ones((16,), jnp.float32)

    _zero_shared_slab(zbuf, acc_sh, 128, NR // (128 * NS))
    plsc.subcore_barrier()

    pltpu.sync_copy(dst_hbm.at[wid], idx_v)

    @pl.loop(0, EK)
    def _(j):
        pltpu.sync_copy(ones_v, acc_sh.at[idx_v.at[j]], add=True)

    plsc.subcore_barrier()
    rows = NR // NS
    pltpu.sync_copy(acc_sh.at[pl.ds(sid * rows, rows)],
                    out_hbm.at[cid, pl.ds(sid * rows, rows)])


@functools.partial(
    pl.kernel,
    out_type=jax.ShapeDtypeStruct((NC, NR, HID), jnp.bfloat16),
    mesh=_vmesh,
    scratch_types=[
        pltpu.VMEM((EK, EC), jnp.int32),
        pltpu.VMEM((EK, EC), jnp.int32),
        pltpu.VMEM((EC, HID), jnp.bfloat16),
        pltpu.VMEM((EC, HID), jnp.bfloat16),
        pltpu.VMEM((EC, HID), jnp.bfloat16),
        pltpu.VMEM((EC, HID), jnp.bfloat16),
        pltpu.VMEM((128, HID), jnp.bfloat16),
        pltpu.VMEM_SHARED((NR, HID), jnp.bfloat16),
        pltpu.VMEM_SHARED((N_NODES, HID), jnp.bfloat16),
        pltpu.SemaphoreType.DMA,
    ],
    compiler_params=_sc_params,
)
def _edge_sc(hp_hbm, src_hbm, dst_hbm, out_hbm,
             idxs_v, idxd_v, rows0, rows1, rows2, rows3, zbuf, acc_sh, hp_sh,
             sem):
    cid = lax.axis_index("c")
    sid = lax.axis_index("s")
    wid = _worker_id()

    _zero_shared_slab(zbuf, acc_sh, 128, NR // (128 * NS))
    hrows = N_NODES // NS
    pltpu.sync_copy(hp_hbm.at[pl.ds(sid * hrows, hrows)],
                    hp_sh.at[pl.ds(sid * hrows, hrows)])
    plsc.subcore_barrier()

    pltpu.sync_copy(src_hbm.at[wid], idxs_v)
    pltpu.sync_copy(dst_hbm.at[wid], idxd_v)

    pltpu.async_copy(hp_sh.at[idxs_v.at[0]], rows0, sem)

    @pl.loop(0, EK, step=2)
    def _(j):
        pltpu.make_async_copy(hp_sh.at[idxs_v.at[j]], rows0, sem).wait()
        pltpu.async_copy(hp_sh.at[idxs_v.at[j + 1]], rows1, sem)
        pltpu.sync_copy(rows0, acc_sh.at[idxd_v.at[j]], add=True)

        pltpu.make_async_copy(hp_sh.at[idxs_v.at[j + 1]], rows1, sem).wait()

        @pl.when(j + 2 < EK)
        def _():
            pltpu.async_copy(hp_sh.at[idxs_v.at[j + 2]], rows0, sem)

        pltpu.sync_copy(rows1, acc_sh.at[idxd_v.at[j + 1]], add=True)

    plsc.subcore_barrier()
    rows = NR // NS
    pltpu.sync_copy(acc_sh.at[pl.ds(sid * rows, rows)],
                    out_hbm.at[cid, pl.ds(sid * rows, rows)])


def _k1_body(dp_ref, x_ref, w_ref, hp_ref, dinv_ref):
    deg = dp_ref[0, 0:N_NODES, 0:1] + dp_ref[1, 0:N_NODES, 0:1] + 1.0
    dinv = lax.rsqrt(deg)
    dinv_ref[...] = dinv
    hp_ref[...] = (jnp.dot(x_ref[...], w_ref[...],
                           preferred_element_type=jnp.float32)
                   * dinv).astype(jnp.bfloat16)


_k1 = pl.pallas_call(
    _k1_body,
    out_shape=(
        jax.ShapeDtypeStruct((N_NODES, HID), jnp.bfloat16),
        jax.ShapeDtypeStruct((N_NODES, 1), jnp.float32),
    ),
)


def _gcn_bn_relu(accp_ref, hp_ref, dinv_ref, b_ref, g_ref, be_ref):
    acc = (accp_ref[0, 0:N_NODES, :].astype(jnp.float32)
           + accp_ref[1, 0:N_NODES, :].astype(jnp.float32)
           + hp_ref[...].astype(jnp.float32))
    gcn = acc * dinv_ref[...] + b_ref[...]
    m = jnp.mean(gcn, axis=0, keepdims=True)
    c = gcn - m
    v = jnp.mean(c * c, axis=0, keepdims=True)
    return jnp.maximum(g_ref[...] * c * lax.rsqrt(v + 1e-5) + be_ref[...], 0.0)


def _post_body(accp_ref, hp_ref, dinv_ref, b_ref, g_ref, be_ref, wn_ref,
               o_ref):
    h = _gcn_bn_relu(accp_ref, hp_ref, dinv_ref, b_ref, g_ref, be_ref)
    o_ref[...] = (jnp.dot(h, wn_ref[...], preferred_element_type=jnp.float32)
                  * dinv_ref[...]).astype(jnp.bfloat16)


_post = pl.pallas_call(
    _post_body,
    out_shape=jax.ShapeDtypeStruct((N_NODES, HID), jnp.bfloat16),
)


def _post3_body(accp_ref, hp_ref, dinv_ref, b_ref, g_ref, be_ref, o_ref):
    h = _gcn_bn_relu(accp_ref, hp_ref, dinv_ref, b_ref, g_ref, be_ref)
    o_ref[...] = jnp.concatenate(
        [h, jnp.ones((N_NODES, 16), jnp.float32)], axis=1)


_post3 = pl.pallas_call(
    _post3_body,
    out_shape=jax.ShapeDtypeStruct((N_NODES, POOL_W), jnp.float32),
)


def _head_body(h_ref, batch_ref, w1_ref, b1_ref, w2t_ref, b2_ref, o_ref):
    gids = lax.broadcasted_iota(jnp.int32, (NUM_GRAPHS, N_NODES), 0)
    mask = (batch_ref[...] == gids).astype(jnp.float32)
    s = jnp.dot(mask, h_ref[...], preferred_element_type=jnp.float32)
    sums = s[:, 0:HID]
    counts = s[:, HID:HID + 1]
    pooled = sums / jnp.maximum(counts, 1.0)
    z = jnp.maximum(
        jnp.dot(pooled, w1_ref[...], preferred_element_type=jnp.float32)
        + b1_ref[...], 0.0)
    o_ref[...] = lax.dot_general(
        w2t_ref[...], z, (((1,), (1,)), ((), ())),
        preferred_element_type=jnp.float32) + b2_ref[...]


_head = pl.pallas_call(
    _head_body,
    out_shape=jax.ShapeDtypeStruct((N_HEADS, NUM_GRAPHS), jnp.float32),
)


def kernel(x, edge_index, edge_attr, batch,
           W1, b1, g1, be1, W2, b2, g2, be2, W3, b3, g3, be3,
           Wh1, bh1, Wh2, bh2):
    del edge_attr

    src = edge_index[0]
    dst = edge_index[1]
    srcp = jnp.pad(src, (0, E_PAD - N_EDGES)).reshape(NW, EK, EC)
    dstp = jnp.pad(dst, (0, E_PAD - N_EDGES),
                   constant_values=GARBAGE_ROW).reshape(NW, EK, EC)

    b1r, g1r, be1r = b1.reshape(1, HID), g1.reshape(1, HID), be1.reshape(1, HID)
    b2r, g2r, be2r = b2.reshape(1, HID), g2.reshape(1, HID), be2.reshape(1, HID)
    b3r, g3r, be3r = b3.reshape(1, HID), g3.reshape(1, HID), be3.reshape(1, HID)
    w1h = Wh1.transpose(1, 0, 2).reshape(HID, N_HEADS * 32)
    b1h = bh1.reshape(1, N_HEADS * 32)
    w2 = Wh2[:, :, 0]
    w2t = (jnp.eye(N_HEADS, dtype=jnp.float32)[:, :, None]
           * w2[:, None, :]).reshape(N_HEADS, N_HEADS * 32)
    b2h = bh2[:, 0:1]

    degp = _deg_sc(dstp)
    h1p, dinv = _k1(degp, x, W1)

    acc1 = _edge_sc(h1p, srcp, dstp)
    h2p = _post(acc1, h1p, dinv, b1r, g1r, be1r, W2)
    acc2 = _edge_sc(h2p, srcp, dstp)
    h3p = _post(acc2, h2p, dinv, b2r, g2r, be2r, W3)
    acc3 = _edge_sc(h3p, srcp, dstp)
    h3aug = _post3(acc3, h3p, dinv, b3r, g3r, be3r)

    return _head(h3aug, batch.reshape(1, N_NODES), w1h, b1h, w2t, b2h)
